# baseline (speedup 1.0000x reference)
import functools

import jax
import jax.numpy as jnp
from jax import lax
from jax.experimental import pallas as pl
from jax.experimental.pallas import tpu as pltpu
from jax.experimental.pallas import tpu_sc as plsc

VOCAB = 1000000
EMB = 64
BATCH = 16384

_INFO = plsc.get_sparse_core_info()
_NC = _INFO.num_cores
_NS = _INFO.num_subcores
_NW = _NC * _NS
_L = 16
_BPW = BATCH // _NW
_GROUPS = _BPW // _L


def _rsqrt_newton(x):
    i = plsc.bitcast(x, jnp.int32)
    magic = jnp.full((_L,), 0x5F3759DF, jnp.int32)
    y = plsc.bitcast(magic - lax.shift_right_logical(i, 1), jnp.float32)
    for _ in range(3):
        y = y * (1.5 - 0.5 * x * y * y)
    return y


def _sc_body(table_hbm, x0_hbm, x1_hbm, out_hbm,
             idx0_v, idx1_v, t_rows, c_rows, out_v, sem0, sem1):
    wid = lax.axis_index("s") * _NC + lax.axis_index("c")
    base = wid * _BPW

    pltpu.sync_copy(x0_hbm.at[pl.ds(base, _BPW)], idx0_v)
    pltpu.sync_copy(x1_hbm.at[pl.ds(base, _BPW)], idx1_v)
    cp0 = pltpu.async_copy(table_hbm.at[idx0_v], t_rows, sem0)
    cp1 = pltpu.async_copy(table_hbm.at[idx1_v], c_rows, sem1)
    cp0.wait()
    cp1.wait()

    def group(g, _):
        rows = jnp.full((_L,), g * _L, jnp.int32) + lax.iota(jnp.int32, _L)

        def step(k, carry):
            dot, na, nb = carry
            col = jnp.full((_L,), k, jnp.int32)
            tv = plsc.load_gather(t_rows, [rows, col])
            cv = plsc.load_gather(c_rows, [rows, col])
            return (dot + tv * cv, na + tv * tv, nb + cv * cv)

        zero = jnp.zeros((_L,), jnp.float32)
        dot, na, nb = lax.fori_loop(0, EMB, step, (zero, zero, zero))
        out_v[pl.ds(g * _L, _L)] = dot * _rsqrt_newton(na * nb)
        return 0

    lax.fori_loop(0, _GROUPS, group, 0)
    pltpu.sync_copy(out_v, out_hbm.at[pl.ds(base, _BPW)])


@functools.partial(
    pl.kernel,
    out_type=jax.ShapeDtypeStruct((BATCH,), jnp.float32),
    mesh=plsc.VectorSubcoreMesh(core_axis_name="c", subcore_axis_name="s"),
    scratch_types=[
        pltpu.VMEM((_BPW,), jnp.int32),
        pltpu.VMEM((_BPW,), jnp.int32),
        pltpu.VMEM((_BPW, EMB), jnp.float32),
        pltpu.VMEM((_BPW, EMB), jnp.float32),
        pltpu.VMEM((_BPW,), jnp.float32),
        pltpu.SemaphoreType.DMA,
        pltpu.SemaphoreType.DMA,
    ],
    compiler_params=pltpu.CompilerParams(
        needs_layout_passes=False, use_tc_tiling_on_sc=False),
)
def _cosine_sc(table_hbm, x0_hbm, x1_hbm, out_hbm,
               idx0_v, idx1_v, t_rows, c_rows, out_v, sem0, sem1):
    _sc_body(table_hbm, x0_hbm, x1_hbm, out_hbm,
             idx0_v, idx1_v, t_rows, c_rows, out_v, sem0, sem1)


def kernel(x, table):
    x0 = jnp.asarray(x[:, 0], jnp.int32)
    x1 = jnp.asarray(x[:, 1], jnp.int32)
    out = _cosine_sc(table, x0, x1)
    return out.reshape(BATCH, 1)


# speedup vs baseline: 1.6403x; 1.6403x over previous
import functools

import jax
import jax.numpy as jnp
from jax import lax
from jax.experimental import pallas as pl
from jax.experimental.pallas import tpu as pltpu
from jax.experimental.pallas import tpu_sc as plsc

VOCAB = 1000000
EMB = 64
BATCH = 16384

_INFO = plsc.get_sparse_core_info()
_NC = _INFO.num_cores
_NS = _INFO.num_subcores
_NW = _NC * _NS
_L = 16
_BPW = BATCH // _NW
_CH = 256


def _rsqrt_newton(x):
    i = plsc.bitcast(x, jnp.int32)
    magic = jnp.full((_L,), 0x5F3759DF, jnp.int32)
    y = plsc.bitcast(magic - lax.shift_right_logical(i, 1), jnp.float32)
    for _ in range(3):
        y = y * (1.5 - 0.5 * x * y * y)
    return y


def _sc_body(table_hbm, x0_hbm, x1_hbm, out_hbm,
             idx0_v, idx1_v, t_rows, c_rows, out_v, sem0, sem1):
    wid = lax.axis_index("s") * _NC + lax.axis_index("c")
    base = wid * _BPW

    pltpu.sync_copy(x0_hbm.at[pl.ds(base, _BPW)], idx0_v)
    pltpu.sync_copy(x1_hbm.at[pl.ds(base, _BPW)], idx1_v)

    for c in range(_BPW // _CH):
        coff = c * _CH

        def issue(gi, _):
            v0 = idx0_v[pl.ds(coff + gi * _L, _L)]
            v1 = idx1_v[pl.ds(coff + gi * _L, _L)]
            for j in range(_L):
                pltpu.async_copy(table_hbm.at[pl.ds(v0[j], 1)],
                                 t_rows.at[pl.ds(gi * _L + j, 1)], sem0)
                pltpu.async_copy(table_hbm.at[pl.ds(v1[j], 1)],
                                 c_rows.at[pl.ds(gi * _L + j, 1)], sem1)
            return 0

        lax.fori_loop(0, _CH // _L, issue, 0)
        pltpu.make_async_copy(table_hbm.at[pl.ds(0, _CH)], t_rows,
                              sem0).wait()
        pltpu.make_async_copy(table_hbm.at[pl.ds(0, _CH)], c_rows,
                              sem1).wait()

        def group(g, _):
            rows = jnp.full((_L,), g * _L, jnp.int32) + lax.iota(jnp.int32, _L)

            def step(k, carry):
                dot, na, nb = carry
                col = jnp.full((_L,), k, jnp.int32)
                tv = plsc.load_gather(t_rows, [rows, col])
                cv = plsc.load_gather(c_rows, [rows, col])
                return (dot + tv * cv, na + tv * tv, nb + cv * cv)

            zero = jnp.zeros((_L,), jnp.float32)
            dot, na, nb = lax.fori_loop(0, EMB, step, (zero, zero, zero))
            out_v[pl.ds(coff + g * _L, _L)] = dot * _rsqrt_newton(na * nb)
            return 0

        lax.fori_loop(0, _CH // _L, group, 0)

    pltpu.sync_copy(out_v, out_hbm.at[pl.ds(base, _BPW)])


@functools.partial(
    pl.kernel,
    out_type=jax.ShapeDtypeStruct((BATCH,), jnp.float32),
    mesh=plsc.VectorSubcoreMesh(core_axis_name="c", subcore_axis_name="s"),
    scratch_types=[
        pltpu.VMEM((_BPW,), jnp.int32),
        pltpu.VMEM((_BPW,), jnp.int32),
        pltpu.VMEM((_CH, EMB), jnp.float32),
        pltpu.VMEM((_CH, EMB), jnp.float32),
        pltpu.VMEM((_BPW,), jnp.float32),
        pltpu.SemaphoreType.DMA,
        pltpu.SemaphoreType.DMA,
    ],
    compiler_params=pltpu.CompilerParams(needs_layout_passes=False),
)
def _cosine_sc(table_hbm, x0_hbm, x1_hbm, out_hbm,
               idx0_v, idx1_v, t_rows, c_rows, out_v, sem0, sem1):
    _sc_body(table_hbm, x0_hbm, x1_hbm, out_hbm,
             idx0_v, idx1_v, t_rows, c_rows, out_v, sem0, sem1)


def kernel(x, table):
    x0 = jnp.asarray(x[:, 0], jnp.int32)
    x1 = jnp.asarray(x[:, 1], jnp.int32)
    out = _cosine_sc(table, x0, x1)
    return out.reshape(BATCH, 1)
